# baseline (speedup 1.0000x reference)
import functools

import jax
import jax.numpy as jnp
from jax import lax
from jax.experimental import pallas as pl
from jax.experimental.pallas import tpu as pltpu
from jax.experimental.pallas import tpu_sc as plsc

B, S, F, I, K, E, TOPK, C = 1, 2048, 768, 768, 5, 8, 2, 256
BETA = 0.5
ST = 256
NS = S // ST
EPAD = 128
U = 2 * S
NT = U // ST
NW = NT + E - 1


def _top2_parts(logits):
    lane = lax.broadcasted_iota(jnp.int32, logits.shape, 1)
    masked = jnp.where(lane < E, logits, -1e30)
    i1 = jnp.argmax(masked, axis=1, keepdims=True)
    v1 = jnp.max(masked, axis=1, keepdims=True)
    masked2 = jnp.where(lane == i1, -1e30, masked)
    i2 = jnp.argmax(masked2, axis=1, keepdims=True)
    v2 = jnp.max(masked2, axis=1, keepdims=True)
    s1 = jax.nn.sigmoid(v1 - v2)
    oh1 = (lane == i1).astype(jnp.float32)
    oh2 = (lane == i2).astype(jnp.float32)
    return oh1, oh2, s1, 1.0 - s1


def _moe_in_kernel(inp_ref, emb_hi_ref, gw_ref, gb_ref, w_ref, out_ref):
    col = inp_ref[...]
    lane = lax.broadcasted_iota(jnp.int32, (ST, C), 1)
    onehot = (col == lane).astype(jnp.float32)
    h = jnp.dot(onehot, emb_hi_ref[...], preferred_element_type=jnp.float32)
    logits = jnp.dot(h, gw_ref[...], preferred_element_type=jnp.float32) + gb_ref[...]
    oh1, oh2, s1, s2 = _top2_parts(logits)
    comb = oh1 * s1 + oh2 * s2
    acc = jnp.zeros((ST, I), jnp.float32)
    for e in range(E):
        ye = lax.dot_general(h, w_ref[e], (((1,), (1,)), ((), ())),
                             preferred_element_type=jnp.float32)
        acc = acc + comb[:, e:e + 1] * ye
    out_ref[...] = jnp.maximum(acc, 0.0)


def _conv_kernel(h1p_ref, wk_ref, gw_ref, gb_ref, ha_ref, hb_ref, inda_ref, indb_ref):
    i = pl.program_id(0)
    blk = h1p_ref[pl.ds(i * ST, ST + 8), :]
    acc = jnp.zeros((ST, I), jnp.float32)
    for kk in range(K):
        xs = lax.slice(blk, (4 + kk, 0), (4 + kk + ST, I))
        acc = acc + lax.dot_general(xs, wk_ref[kk], (((1,), (1,)), ((), ())),
                                    preferred_element_type=jnp.float32)
    h2 = jnp.maximum(acc, 0.0)
    logits = jnp.dot(h2, gw_ref[...], preferred_element_type=jnp.float32) + gb_ref[...]
    oh1, oh2, s1, s2 = _top2_parts(logits)
    ha_ref[...] = h2 * s1
    hb_ref[...] = h2 * s2
    inda_ref[...] = oh1
    indb_ref[...] = oh2


def _route_kernel(ind_ref, dest_ref, meta_ref, rank_ref):
    f32 = jnp.float32
    r = lax.broadcasted_iota(jnp.int32, (ST, ST), 0)
    c = lax.broadcasted_iota(jnp.int32, (ST, ST), 1)
    tri = (c < r).astype(f32)
    run = jnp.zeros((1, EPAD), f32)
    for ch in range(U // ST):
        ind_c = ind_ref[ch * ST:(ch + 1) * ST, :]
        rank_c = jnp.dot(tri, ind_c, preferred_element_type=f32) + run
        rank_ref[ch * ST:(ch + 1) * ST, :] = rank_c
        run = run + jnp.sum(ind_c, axis=0, keepdims=True)
    counts = run
    re = lax.broadcasted_iota(jnp.int32, (EPAD, EPAD), 0)
    ce = lax.broadcasted_iota(jnp.int32, (EPAD, EPAD), 1)
    trie = (re < ce).astype(f32)
    offs = jnp.dot(counts, trie, preferred_element_type=f32)
    for ch in range(U // ST):
        ind_c = ind_ref[ch * ST:(ch + 1) * ST, :]
        d = jnp.sum(ind_c * (rank_ref[ch * ST:(ch + 1) * ST, :] + offs),
                    axis=1, keepdims=True)
        dest_ref[ch * ST:(ch + 1) * ST, :] = d.astype(jnp.int32)
    jv = lax.broadcasted_iota(jnp.int32, (NT, 1), 0).astype(f32)
    tile_lo = jv * ST
    tile_hi = tile_lo + ST
    lo_e = offs
    hi_e = offs + counts
    flag = ((lo_e < tile_hi) & (hi_e > tile_lo) & (counts > 0.0)).astype(f32)
    rowsum = jnp.sum(flag, axis=1, keepdims=True)
    rj = lax.broadcasted_iota(jnp.int32, (NT, NT), 0)
    cj = lax.broadcasted_iota(jnp.int32, (NT, NT), 1)
    trij = (cj < rj).astype(f32)
    prevrows = jnp.dot(trij, rowsum, preferred_element_type=f32)
    excl_e = jnp.dot(flag, trie, preferred_element_type=f32)
    widx = prevrows + excl_e
    first = flag * (excl_e == 0.0).astype(f32)
    ev = lax.broadcasted_iota(jnp.int32, (1, EPAD), 1).astype(f32)
    wlane = lax.broadcasted_iota(jnp.int32, (1, EPAD), 1)
    rows = []
    wt_row = jnp.zeros((1, EPAD), f32)
    we_row = jnp.zeros((1, EPAD), f32)
    wl_row = jnp.zeros((1, EPAD), f32)
    wh_row = jnp.zeros((1, EPAD), f32)
    wf_row = jnp.zeros((1, EPAD), f32)
    for w in range(NW):
        sel = flag * (widx == float(w)).astype(f32)
        has = jnp.sum(sel)
        wt = jnp.sum(sel * jv) + (1.0 - has) * float(NT - 1)
        we = jnp.sum(sel * ev)
        wl = jnp.sum(sel * jnp.maximum(lo_e, tile_lo))
        wh = jnp.sum(sel * jnp.minimum(hi_e, tile_hi))
        wf = jnp.sum(sel * first)
        oh = (wlane == w).astype(f32)
        wt_row = wt_row + oh * wt
        we_row = we_row + oh * we
        wl_row = wl_row + oh * wl
        wh_row = wh_row + oh * wh
        wf_row = wf_row + oh * wf
    z = jnp.zeros((1, EPAD), f32)
    meta = jnp.concatenate([wt_row, we_row, wl_row, wh_row, wf_row, z, z, z], axis=0)
    meta_ref[...] = meta.astype(jnp.int32)


def _group_kernel(wt_ref, we_ref, wl_ref, wh_ref, wf_ref, h_ref, w_ref, out_ref):
    w = pl.program_id(1)
    rows = wt_ref[w] * ST + lax.broadcasted_iota(jnp.int32, (ST, 1), 0)
    mask = ((rows >= wl_ref[w]) & (rows < wh_ref[w])).astype(jnp.float32)
    hm = h_ref[...] * mask
    y = lax.dot_general(hm, w_ref[0], (((1,), (1,)), ((), ())),
                        preferred_element_type=jnp.float32)

    @pl.when(wf_ref[w] == 1)
    def _():
        out_ref[...] = y

    @pl.when(wf_ref[w] != 1)
    def _():
        out_ref[...] += y


def _post_kernel(o_ref, inp_ref, tgt_ref, emb_ref, owt_ref, ob_ref, out_ref):
    i = pl.program_id(0)
    o = o_ref[...]
    d, sc, sh = o[:, :F], o[:, F:2 * F], o[:, 2 * F:]
    r = lax.broadcasted_iota(jnp.int32, (F, F), 0)
    c = lax.broadcasted_iota(jnp.int32, (F, F), 1)
    tri = (r <= c).astype(jnp.float32)
    cum = jnp.dot(d, tri, preferred_element_type=jnp.float32)
    pos = (i * ST + lax.broadcasted_iota(jnp.int32, (ST, 1), 0)).astype(jnp.float32)
    y = cum / (pos + 1.0) * sc + sh
    y = y - jnp.mean(y, axis=1, keepdims=True)
    nrm = jnp.sqrt(jnp.sum(y * y, axis=1, keepdims=True))
    y = y / (nrm * (F ** -0.5) + 1e-5)
    col = inp_ref[...]
    lane = lax.broadcasted_iota(jnp.int32, (ST, C), 1)
    onehot = (col == lane).astype(jnp.float32)
    x = jnp.dot(onehot, emb_ref[...], preferred_element_type=jnp.float32)
    x0, x1 = x[:, :F], x[:, F:]
    y1 = x0 * BETA + y * (1.0 - BETA)
    y2 = x1 + y1
    cat = jnp.concatenate([y1, y2], axis=1)
    logits = jnp.dot(cat, owt_ref[...], preferred_element_type=jnp.float32) + ob_ref[...]
    m = jnp.max(logits, axis=1, keepdims=True)
    lse = m + jnp.log(jnp.sum(jnp.exp(logits - m), axis=1, keepdims=True))
    tcol = tgt_ref[...]
    tsel = (tcol == lane).astype(jnp.float32)
    g = jnp.sum(logits * tsel, axis=1, keepdims=True)
    part = jnp.sum(lse - g, keepdims=True).reshape(1, 1)

    @pl.when(i == 0)
    def _():
        out_ref[...] = jnp.zeros_like(out_ref)

    out_ref[...] += part


def _make_sc_kernels():
    mesh = plsc.VectorSubcoreMesh(core_axis_name="c", subcore_axis_name="s")
    nc, ns = mesh.num_cores, mesh.num_subcores
    nw = nc * ns
    i32, f32 = jnp.int32, jnp.float32

    g_rows = U // nw

    @functools.partial(
        pl.kernel, out_type=jax.ShapeDtypeStruct((U, I), f32), mesh=mesh,
        scratch_types=[pltpu.VMEM((g_rows,), i32), pltpu.VMEM((g_rows, I), f32),
                       pltpu.SemaphoreType.DMA])
    def dispatch_sc(src_hbm, dest_hbm, out_hbm, idx_v, rows_v, sem):
        wid = lax.axis_index("s") * nc + lax.axis_index("c")
        base = wid * g_rows
        pltpu.sync_copy(dest_hbm.at[pl.ds(base, g_rows)], idx_v)
        pltpu.sync_copy(src_hbm.at[pl.ds(base, g_rows)], rows_v)
        pltpu.async_copy(rows_v, out_hbm.at[idx_v], sem).wait()

    t_per_w = S // nw
    CH = 16
    D3 = 3 * F

    @functools.partial(
        pl.kernel, out_type=jax.ShapeDtypeStruct((S, D3), f32), mesh=mesh,
        scratch_types=[pltpu.VMEM((t_per_w,), i32), pltpu.VMEM((t_per_w,), i32),
                       pltpu.VMEM((CH, D3), f32), pltpu.VMEM((CH, D3), f32),
                       pltpu.SemaphoreType.DMA])
    def combine_sc(ysort_hbm, dest_hbm, out_hbm, idx1_v, idx2_v, r1_v, r2_v, sem):
        wid = lax.axis_index("s") * nc + lax.axis_index("c")
        base = wid * t_per_w
        pltpu.sync_copy(dest_hbm.at[pl.ds(base, t_per_w)], idx1_v)
        pltpu.sync_copy(dest_hbm.at[pl.ds(S + base, t_per_w)], idx2_v)

        def chunk(ch, carry):
            i1 = idx1_v[pl.ds(ch * CH, CH)]
            i2 = idx2_v[pl.ds(ch * CH, CH)]
            pltpu.async_copy(ysort_hbm.at[i1], r1_v, sem).wait()
            pltpu.async_copy(ysort_hbm.at[i2], r2_v, sem).wait()

            def row(rr, c2):
                def qcol(q, c3):
                    a = r1_v[rr, pl.ds(q * 16, 16)]
                    b = r2_v[rr, pl.ds(q * 16, 16)]
                    r1_v[rr, pl.ds(q * 16, 16)] = a + b
                    return c3

                lax.fori_loop(0, D3 // 16, qcol, 0)
                return c2

            lax.fori_loop(0, CH, row, 0)
            pltpu.sync_copy(r1_v, out_hbm.at[pl.ds(base + ch * CH, CH)])
            return carry

        lax.fori_loop(0, t_per_w // CH, chunk, 0)

    return dispatch_sc, combine_sc


def kernel(inp, tgt, emb, gate_w_in, gate_b_in, w_moe_in, w1, gate_w_out, gate_b_out, w_moe_out, out_w, out_b):
    f32, i32 = jnp.float32, jnp.int32
    inp2 = inp.reshape(S, 1).astype(i32)
    tgt2 = tgt.reshape(S, 1).astype(i32)
    emb_hi = emb[:, F:]
    gw_in = jnp.zeros((F, EPAD), f32).at[:, :E].set(gate_w_in)
    gb_in = jnp.zeros((1, EPAD), f32).at[0, :E].set(gate_b_in)
    gw_out = jnp.zeros((I, EPAD), f32).at[:, :E].set(gate_w_out)
    gb_out = jnp.zeros((1, EPAD), f32).at[0, :E].set(gate_b_out)
    wk = jnp.transpose(w1, (2, 0, 1))
    owt = out_w.T
    obr = out_b.reshape(1, C)

    h1 = pl.pallas_call(
        _moe_in_kernel,
        grid=(NS,),
        in_specs=[
            pl.BlockSpec((ST, 1), lambda i: (i, 0)),
            pl.BlockSpec((C, F), lambda i: (0, 0)),
            pl.BlockSpec((F, EPAD), lambda i: (0, 0)),
            pl.BlockSpec((1, EPAD), lambda i: (0, 0)),
            pl.BlockSpec((E, I, F), lambda i: (0, 0, 0)),
        ],
        out_specs=pl.BlockSpec((ST, I), lambda i: (i, 0)),
        out_shape=jax.ShapeDtypeStruct((S, I), f32),
    )(inp2, emb_hi, gw_in, gb_in, w_moe_in)

    h1p = jnp.zeros((S + 8, I), f32).at[8:].set(h1)

    ha, hb, inda, indb = pl.pallas_call(
        _conv_kernel,
        grid=(NS,),
        in_specs=[
            pl.BlockSpec((S + 8, I), lambda i: (0, 0)),
            pl.BlockSpec((K, I, I), lambda i: (0, 0, 0)),
            pl.BlockSpec((I, EPAD), lambda i: (0, 0)),
            pl.BlockSpec((1, EPAD), lambda i: (0, 0)),
        ],
        out_specs=[
            pl.BlockSpec((ST, I), lambda i: (i, 0)),
            pl.BlockSpec((ST, I), lambda i: (i, 0)),
            pl.BlockSpec((ST, EPAD), lambda i: (i, 0)),
            pl.BlockSpec((ST, EPAD), lambda i: (i, 0)),
        ],
        out_shape=[
            jax.ShapeDtypeStruct((S, I), f32),
            jax.ShapeDtypeStruct((S, I), f32),
            jax.ShapeDtypeStruct((S, EPAD), f32),
            jax.ShapeDtypeStruct((S, EPAD), f32),
        ],
    )(h1p, wk, gw_out, gb_out)

    h2s = jnp.concatenate([ha, hb], axis=0)
    ind = jnp.concatenate([inda, indb], axis=0)

    dest2d, meta = pl.pallas_call(
        _route_kernel,
        grid=(1,),
        in_specs=[pl.BlockSpec((U, EPAD), lambda i: (0, 0))],
        out_specs=[
            pl.BlockSpec((U, 1), lambda i: (0, 0)),
            pl.BlockSpec((8, EPAD), lambda i: (0, 0)),
        ],
        out_shape=[
            jax.ShapeDtypeStruct((U, 1), i32),
            jax.ShapeDtypeStruct((8, EPAD), i32),
        ],
        scratch_shapes=[pltpu.VMEM((U, EPAD), f32)],
    )(ind)

    dest = dest2d.reshape(U)
    wt, we, wl, wh, wf = (meta[0, :NW], meta[1, :NW], meta[2, :NW],
                          meta[3, :NW], meta[4, :NW])

    dispatch_sc, combine_sc = _make_sc_kernels()
    hsort = dispatch_sc(h2s, dest)

    ysort = pl.pallas_call(
        _group_kernel,
        grid_spec=pltpu.PrefetchScalarGridSpec(
            num_scalar_prefetch=5,
            grid=(3, NW),
            in_specs=[
                pl.BlockSpec((ST, I), lambda o, w, wt, we, wl, wh, wf: (wt[w], 0)),
                pl.BlockSpec((1, F, I), lambda o, w, wt, we, wl, wh, wf: (we[w], o, 0)),
            ],
            out_specs=pl.BlockSpec(
                (ST, F), lambda o, w, wt, we, wl, wh, wf: (wt[w], o)),
        ),
        out_shape=jax.ShapeDtypeStruct((U, 3 * F), f32),
    )(wt, we, wl, wh, wf, hsort, w_moe_out)

    o_out = combine_sc(ysort, dest)

    tot = pl.pallas_call(
        _post_kernel,
        grid=(NS,),
        in_specs=[
            pl.BlockSpec((ST, 3 * F), lambda i: (i, 0)),
            pl.BlockSpec((ST, 1), lambda i: (i, 0)),
            pl.BlockSpec((ST, 1), lambda i: (i, 0)),
            pl.BlockSpec((C, 2 * F), lambda i: (0, 0)),
            pl.BlockSpec((2 * F, C), lambda i: (0, 0)),
            pl.BlockSpec((1, C), lambda i: (0, 0)),
        ],
        out_specs=pl.BlockSpec((1, 1), lambda i: (0, 0)),
        out_shape=jax.ShapeDtypeStruct((1, 1), f32),
    )(o_out, inp2, tgt2, emb, owt, obr)

    return tot[0, 0] / float(B * S)


# speedup vs baseline: 1.2944x; 1.2944x over previous
import functools

import jax
import jax.numpy as jnp
from jax import lax
from jax.experimental import pallas as pl
from jax.experimental.pallas import tpu as pltpu
from jax.experimental.pallas import tpu_sc as plsc

B, S, F, I, K, E, TOPK, C = 1, 2048, 768, 768, 5, 8, 2, 256
BETA = 0.5
ST = 256
NS = S // ST
EPAD = 128
U = 2 * S
NT = U // ST
NW = NT + E - 1


def _top2_parts(logits):
    lane = lax.broadcasted_iota(jnp.int32, logits.shape, 1)
    masked = jnp.where(lane < E, logits, -1e30)
    i1 = jnp.argmax(masked, axis=1, keepdims=True)
    v1 = jnp.max(masked, axis=1, keepdims=True)
    masked2 = jnp.where(lane == i1, -1e30, masked)
    i2 = jnp.argmax(masked2, axis=1, keepdims=True)
    v2 = jnp.max(masked2, axis=1, keepdims=True)
    s1 = jax.nn.sigmoid(v1 - v2)
    oh1 = (lane == i1).astype(jnp.float32)
    oh2 = (lane == i2).astype(jnp.float32)
    return oh1, oh2, s1, 1.0 - s1


def _moe_in_kernel(inp_ref, emb_hi_ref, gw_ref, gb_ref, w_ref, out_ref):
    col = inp_ref[...]
    lane = lax.broadcasted_iota(jnp.int32, (ST, C), 1)
    onehot = (col == lane).astype(jnp.float32)
    h = jnp.dot(onehot, emb_hi_ref[...], preferred_element_type=jnp.float32)
    logits = jnp.dot(h, gw_ref[...], preferred_element_type=jnp.float32) + gb_ref[...]
    oh1, oh2, s1, s2 = _top2_parts(logits)
    comb = oh1 * s1 + oh2 * s2
    acc = jnp.zeros((ST, I), jnp.float32)
    for e in range(E):
        ye = lax.dot_general(h, w_ref[e], (((1,), (1,)), ((), ())),
                             preferred_element_type=jnp.float32)
        acc = acc + comb[:, e:e + 1] * ye
    out_ref[...] = jnp.maximum(acc, 0.0)


def _conv_kernel(h1p_ref, wk_ref, gw_ref, gb_ref, ha_ref, hb_ref, inda_ref, indb_ref):
    i = pl.program_id(0)
    blk = h1p_ref[pl.ds(i * ST, ST + 8), :]
    acc = jnp.zeros((ST, I), jnp.float32)
    for kk in range(K):
        xs = lax.slice(blk, (4 + kk, 0), (4 + kk + ST, I))
        acc = acc + lax.dot_general(xs, wk_ref[kk], (((1,), (1,)), ((), ())),
                                    preferred_element_type=jnp.float32)
    h2 = jnp.maximum(acc, 0.0)
    logits = jnp.dot(h2, gw_ref[...], preferred_element_type=jnp.float32) + gb_ref[...]
    oh1, oh2, s1, s2 = _top2_parts(logits)
    ha_ref[...] = h2 * s1
    hb_ref[...] = h2 * s2
    inda_ref[...] = oh1
    indb_ref[...] = oh2


def _route_kernel(ind_ref, dest_ref, meta_ref, rank_ref):
    f32 = jnp.float32
    r = lax.broadcasted_iota(jnp.int32, (ST, ST), 0)
    c = lax.broadcasted_iota(jnp.int32, (ST, ST), 1)
    tri = (c < r).astype(f32)
    run = jnp.zeros((1, EPAD), f32)
    for ch in range(U // ST):
        ind_c = ind_ref[ch * ST:(ch + 1) * ST, :]
        rank_c = jnp.dot(tri, ind_c, preferred_element_type=f32) + run
        rank_ref[ch * ST:(ch + 1) * ST, :] = rank_c
        run = run + jnp.sum(ind_c, axis=0, keepdims=True)
    counts = run
    re = lax.broadcasted_iota(jnp.int32, (EPAD, EPAD), 0)
    ce = lax.broadcasted_iota(jnp.int32, (EPAD, EPAD), 1)
    trie = (re < ce).astype(f32)
    offs = jnp.dot(counts, trie, preferred_element_type=f32)
    for ch in range(U // ST):
        ind_c = ind_ref[ch * ST:(ch + 1) * ST, :]
        d = jnp.sum(ind_c * (rank_ref[ch * ST:(ch + 1) * ST, :] + offs),
                    axis=1, keepdims=True)
        dest_ref[ch * ST:(ch + 1) * ST, :] = d.astype(jnp.int32)
    jv = lax.broadcasted_iota(jnp.int32, (NT, 1), 0).astype(f32)
    tile_lo = jv * ST
    tile_hi = tile_lo + ST
    lo_e = offs
    hi_e = offs + counts
    flag = ((lo_e < tile_hi) & (hi_e > tile_lo) & (counts > 0.0)).astype(f32)
    rowsum = jnp.sum(flag, axis=1, keepdims=True)
    rj = lax.broadcasted_iota(jnp.int32, (NT, NT), 0)
    cj = lax.broadcasted_iota(jnp.int32, (NT, NT), 1)
    trij = (cj < rj).astype(f32)
    prevrows = jnp.dot(trij, rowsum, preferred_element_type=f32)
    excl_e = jnp.dot(flag, trie, preferred_element_type=f32)
    widx = prevrows + excl_e
    first = flag * (excl_e == 0.0).astype(f32)
    ev = lax.broadcasted_iota(jnp.int32, (1, EPAD), 1).astype(f32)
    wlane = lax.broadcasted_iota(jnp.int32, (1, EPAD), 1)
    rows = []
    wt_row = jnp.zeros((1, EPAD), f32)
    we_row = jnp.zeros((1, EPAD), f32)
    wl_row = jnp.zeros((1, EPAD), f32)
    wh_row = jnp.zeros((1, EPAD), f32)
    wf_row = jnp.zeros((1, EPAD), f32)
    for w in range(NW):
        sel = flag * (widx == float(w)).astype(f32)
        has = jnp.sum(sel)
        wt = jnp.sum(sel * jv) + (1.0 - has) * float(NT - 1)
        we = jnp.sum(sel * ev)
        wl = jnp.sum(sel * jnp.maximum(lo_e, tile_lo))
        wh = jnp.sum(sel * jnp.minimum(hi_e, tile_hi))
        wf = jnp.sum(sel * first)
        oh = (wlane == w).astype(f32)
        wt_row = wt_row + oh * wt
        we_row = we_row + oh * we
        wl_row = wl_row + oh * wl
        wh_row = wh_row + oh * wh
        wf_row = wf_row + oh * wf
    z = jnp.zeros((1, EPAD), f32)
    meta = jnp.concatenate([wt_row, we_row, wl_row, wh_row, wf_row, z, z, z], axis=0)
    meta_ref[...] = meta.astype(jnp.int32)


def _group_kernel(wt_ref, we_ref, wl_ref, wh_ref, wf_ref, h_ref, w_ref, out_ref):
    w = pl.program_id(0)
    rows = wt_ref[w] * ST + lax.broadcasted_iota(jnp.int32, (ST, 1), 0)
    mask = ((rows >= wl_ref[w]) & (rows < wh_ref[w])).astype(jnp.float32)
    hm = h_ref[...] * mask
    y = lax.dot_general(hm, w_ref[0], (((1,), (1,)), ((), ())),
                        preferred_element_type=jnp.float32)

    @pl.when(wf_ref[w] == 1)
    def _():
        out_ref[...] = y

    @pl.when(wf_ref[w] != 1)
    def _():
        out_ref[...] += y


def _post_kernel(oa_ref, ob2_ref, inp_ref, tgt_ref, emb_ref, owt_ref, ob_ref, out_ref):
    i = pl.program_id(0)
    o = oa_ref[...] + ob2_ref[...]
    d, sc, sh = o[:, :F], o[:, F:2 * F], o[:, 2 * F:]
    r = lax.broadcasted_iota(jnp.int32, (F, F), 0)
    c = lax.broadcasted_iota(jnp.int32, (F, F), 1)
    tri = (r <= c).astype(jnp.float32)
    cum = jnp.dot(d, tri, preferred_element_type=jnp.float32)
    pos = (i * ST + lax.broadcasted_iota(jnp.int32, (ST, 1), 0)).astype(jnp.float32)
    y = cum / (pos + 1.0) * sc + sh
    y = y - jnp.mean(y, axis=1, keepdims=True)
    nrm = jnp.sqrt(jnp.sum(y * y, axis=1, keepdims=True))
    y = y / (nrm * (F ** -0.5) + 1e-5)
    col = inp_ref[...]
    lane = lax.broadcasted_iota(jnp.int32, (ST, C), 1)
    onehot = (col == lane).astype(jnp.float32)
    x = jnp.dot(onehot, emb_ref[...], preferred_element_type=jnp.float32)
    x0, x1 = x[:, :F], x[:, F:]
    y1 = x0 * BETA + y * (1.0 - BETA)
    y2 = x1 + y1
    cat = jnp.concatenate([y1, y2], axis=1)
    logits = jnp.dot(cat, owt_ref[...], preferred_element_type=jnp.float32) + ob_ref[...]
    m = jnp.max(logits, axis=1, keepdims=True)
    lse = m + jnp.log(jnp.sum(jnp.exp(logits - m), axis=1, keepdims=True))
    tcol = tgt_ref[...]
    tsel = (tcol == lane).astype(jnp.float32)
    g = jnp.sum(logits * tsel, axis=1, keepdims=True)
    part = jnp.sum(lse - g, keepdims=True).reshape(1, 1)

    @pl.when(i == 0)
    def _():
        out_ref[...] = jnp.zeros_like(out_ref)

    out_ref[...] += part


def _make_sc_kernels():
    mesh = plsc.VectorSubcoreMesh(core_axis_name="c", subcore_axis_name="s")
    nc, ns = mesh.num_cores, mesh.num_subcores
    nw = nc * ns
    i32, f32 = jnp.int32, jnp.float32

    g_rows = U // nw

    @functools.partial(
        pl.kernel, out_type=jax.ShapeDtypeStruct((U, I), f32), mesh=mesh,
        scratch_types=[pltpu.VMEM((g_rows,), i32), pltpu.VMEM((g_rows, I), f32),
                       pltpu.SemaphoreType.DMA])
    def dispatch_sc(src_hbm, dest_hbm, out_hbm, idx_v, rows_v, sem):
        wid = lax.axis_index("s") * nc + lax.axis_index("c")
        base = wid * g_rows
        pltpu.sync_copy(dest_hbm.at[pl.ds(base, g_rows)], idx_v)
        pltpu.sync_copy(src_hbm.at[pl.ds(base, g_rows)], rows_v)
        pltpu.async_copy(rows_v, out_hbm.at[idx_v], sem).wait()

    u_per_w = U // nw
    CH = 32
    D3 = 3 * F

    @functools.partial(
        pl.kernel, out_type=jax.ShapeDtypeStruct((U, D3), f32), mesh=mesh,
        scratch_types=[pltpu.VMEM((u_per_w,), i32), pltpu.VMEM((CH, D3), f32),
                       pltpu.SemaphoreType.DMA])
    def combine_sc(ysort_hbm, dest_hbm, out_hbm, idx_v, rows_v, sem):
        wid = lax.axis_index("s") * nc + lax.axis_index("c")
        base = wid * u_per_w
        pltpu.sync_copy(dest_hbm.at[pl.ds(base, u_per_w)], idx_v)

        def chunk(ch, carry):
            pltpu.async_copy(
                ysort_hbm.at[idx_v.at[pl.ds(ch * CH, CH)]], rows_v, sem).wait()
            pltpu.sync_copy(rows_v, out_hbm.at[pl.ds(base + ch * CH, CH)])
            return carry

        lax.fori_loop(0, u_per_w // CH, chunk, 0)

    return dispatch_sc, combine_sc


def kernel(inp, tgt, emb, gate_w_in, gate_b_in, w_moe_in, w1, gate_w_out, gate_b_out, w_moe_out, out_w, out_b):
    f32, i32 = jnp.float32, jnp.int32
    inp2 = inp.reshape(S, 1).astype(i32)
    tgt2 = tgt.reshape(S, 1).astype(i32)
    emb_hi = emb[:, F:]
    gw_in = jnp.zeros((F, EPAD), f32).at[:, :E].set(gate_w_in)
    gb_in = jnp.zeros((1, EPAD), f32).at[0, :E].set(gate_b_in)
    gw_out = jnp.zeros((I, EPAD), f32).at[:, :E].set(gate_w_out)
    gb_out = jnp.zeros((1, EPAD), f32).at[0, :E].set(gate_b_out)
    wk = jnp.transpose(w1, (2, 0, 1))
    owt = out_w.T
    obr = out_b.reshape(1, C)

    h1 = pl.pallas_call(
        _moe_in_kernel,
        grid=(NS,),
        in_specs=[
            pl.BlockSpec((ST, 1), lambda i: (i, 0)),
            pl.BlockSpec((C, F), lambda i: (0, 0)),
            pl.BlockSpec((F, EPAD), lambda i: (0, 0)),
            pl.BlockSpec((1, EPAD), lambda i: (0, 0)),
            pl.BlockSpec((E, I, F), lambda i: (0, 0, 0)),
        ],
        out_specs=pl.BlockSpec((ST, I), lambda i: (i, 0)),
        out_shape=jax.ShapeDtypeStruct((S, I), f32),
    )(inp2, emb_hi, gw_in, gb_in, w_moe_in)

    h1p = jnp.zeros((S + 8, I), f32).at[8:].set(h1)

    ha, hb, inda, indb = pl.pallas_call(
        _conv_kernel,
        grid=(NS,),
        in_specs=[
            pl.BlockSpec((S + 8, I), lambda i: (0, 0)),
            pl.BlockSpec((K, I, I), lambda i: (0, 0, 0)),
            pl.BlockSpec((I, EPAD), lambda i: (0, 0)),
            pl.BlockSpec((1, EPAD), lambda i: (0, 0)),
        ],
        out_specs=[
            pl.BlockSpec((ST, I), lambda i: (i, 0)),
            pl.BlockSpec((ST, I), lambda i: (i, 0)),
            pl.BlockSpec((ST, EPAD), lambda i: (i, 0)),
            pl.BlockSpec((ST, EPAD), lambda i: (i, 0)),
        ],
        out_shape=[
            jax.ShapeDtypeStruct((S, I), f32),
            jax.ShapeDtypeStruct((S, I), f32),
            jax.ShapeDtypeStruct((S, EPAD), f32),
            jax.ShapeDtypeStruct((S, EPAD), f32),
        ],
    )(h1p, wk, gw_out, gb_out)

    h2s = jnp.concatenate([ha, hb], axis=0)
    ind = jnp.concatenate([inda, indb], axis=0)

    dest2d, meta = pl.pallas_call(
        _route_kernel,
        grid=(1,),
        in_specs=[pl.BlockSpec((U, EPAD), lambda i: (0, 0))],
        out_specs=[
            pl.BlockSpec((U, 1), lambda i: (0, 0)),
            pl.BlockSpec((8, EPAD), lambda i: (0, 0)),
        ],
        out_shape=[
            jax.ShapeDtypeStruct((U, 1), i32),
            jax.ShapeDtypeStruct((8, EPAD), i32),
        ],
        scratch_shapes=[pltpu.VMEM((U, EPAD), f32)],
    )(ind)

    dest = dest2d.reshape(U)
    wt, we, wl, wh, wf = (meta[0, :NW], meta[1, :NW], meta[2, :NW],
                          meta[3, :NW], meta[4, :NW])

    dispatch_sc, combine_sc = _make_sc_kernels()
    hsort = dispatch_sc(h2s, dest)

    ysort = pl.pallas_call(
        _group_kernel,
        grid_spec=pltpu.PrefetchScalarGridSpec(
            num_scalar_prefetch=5,
            grid=(NW,),
            in_specs=[
                pl.BlockSpec((ST, I), lambda w, wt, we, wl, wh, wf: (wt[w], 0)),
                pl.BlockSpec((1, 3 * F, I), lambda w, wt, we, wl, wh, wf: (we[w], 0, 0)),
            ],
            out_specs=pl.BlockSpec(
                (ST, 3 * F), lambda w, wt, we, wl, wh, wf: (wt[w], 0)),
        ),
        out_shape=jax.ShapeDtypeStruct((U, 3 * F), f32),
    )(wt, we, wl, wh, wf, hsort, w_moe_out)

    o_slots = combine_sc(ysort, dest)

    tot = pl.pallas_call(
        _post_kernel,
        grid=(NS,),
        in_specs=[
            pl.BlockSpec((ST, 3 * F), lambda i: (i, 0)),
            pl.BlockSpec((ST, 3 * F), lambda i: (i + NS, 0)),
            pl.BlockSpec((ST, 1), lambda i: (i, 0)),
            pl.BlockSpec((ST, 1), lambda i: (i, 0)),
            pl.BlockSpec((C, 2 * F), lambda i: (0, 0)),
            pl.BlockSpec((2 * F, C), lambda i: (0, 0)),
            pl.BlockSpec((1, C), lambda i: (0, 0)),
        ],
        out_specs=pl.BlockSpec((1, 1), lambda i: (0, 0)),
        out_shape=jax.ShapeDtypeStruct((1, 1), f32),
    )(o_slots, o_slots, inp2, tgt2, emb, owt, obr)

    return tot[0, 0] / float(B * S)


# speedup vs baseline: 1.3785x; 1.0650x over previous
import functools

import jax
import jax.numpy as jnp
from jax import lax
from jax.experimental import pallas as pl
from jax.experimental.pallas import tpu as pltpu
from jax.experimental.pallas import tpu_sc as plsc

B, S, F, I, K, E, TOPK, C = 1, 2048, 768, 768, 5, 8, 2, 256
BETA = 0.5
ST = 256
NS = S // ST
EPAD = 128
U = 2 * S
NT = U // ST
NW = NT + E - 1


def _top2_parts(logits):
    lane = lax.broadcasted_iota(jnp.int32, logits.shape, 1)
    masked = jnp.where(lane < E, logits, -1e30)
    i1 = jnp.argmax(masked, axis=1, keepdims=True)
    v1 = jnp.max(masked, axis=1, keepdims=True)
    masked2 = jnp.where(lane == i1, -1e30, masked)
    i2 = jnp.argmax(masked2, axis=1, keepdims=True)
    v2 = jnp.max(masked2, axis=1, keepdims=True)
    s1 = jax.nn.sigmoid(v1 - v2)
    oh1 = (lane == i1).astype(jnp.float32)
    oh2 = (lane == i2).astype(jnp.float32)
    return oh1, oh2, s1, 1.0 - s1


def _moe_in_kernel(inp_ref, emb_hi_ref, gw_ref, gb_ref, w_ref, out_ref):
    col = inp_ref[...]
    lane = lax.broadcasted_iota(jnp.int32, (ST, C), 1)
    onehot = (col == lane).astype(jnp.float32)
    h = jnp.dot(onehot, emb_hi_ref[...], preferred_element_type=jnp.float32)
    logits = jnp.dot(h, gw_ref[...], preferred_element_type=jnp.float32) + gb_ref[...]
    oh1, oh2, s1, s2 = _top2_parts(logits)
    comb = oh1 * s1 + oh2 * s2
    acc = jnp.zeros((ST, I), jnp.float32)
    for e in range(E):
        ye = lax.dot_general(h, w_ref[e], (((1,), (1,)), ((), ())),
                             preferred_element_type=jnp.float32)
        acc = acc + comb[:, e:e + 1] * ye
    out_ref[...] = jnp.maximum(acc, 0.0)


def _conv_kernel(h1p_ref, wk_ref, gw_ref, gb_ref, ha_ref, hb_ref, dest_ref, meta_ref, ind_ref, rank_ref):
    i = pl.program_id(0)

    @pl.when(i < NS)
    def _():
        im = jnp.minimum(i, NS - 1)
        blk = h1p_ref[pl.ds(im * ST, ST + 8), :]
        acc = jnp.zeros((ST, I), jnp.float32)
        for kk in range(K):
            xs = lax.slice(blk, (4 + kk, 0), (4 + kk + ST, I))
            acc = acc + lax.dot_general(xs, wk_ref[kk], (((1,), (1,)), ((), ())),
                                        preferred_element_type=jnp.float32)
        h2 = jnp.maximum(acc, 0.0)
        logits = jnp.dot(h2, gw_ref[...], preferred_element_type=jnp.float32) + gb_ref[...]
        oh1, oh2, s1, s2 = _top2_parts(logits)
        ha_ref[...] = h2 * s1
        hb_ref[...] = h2 * s2
        ind_ref[pl.ds(im * ST, ST), :] = oh1
        ind_ref[pl.ds(S + im * ST, ST), :] = oh2

    @pl.when(i == NS)
    def _():
        _route(ind_ref, dest_ref, meta_ref, rank_ref)


def _route(ind_ref, dest_ref, meta_ref, rank_ref):
    f32 = jnp.float32
    r = lax.broadcasted_iota(jnp.int32, (ST, ST), 0)
    c = lax.broadcasted_iota(jnp.int32, (ST, ST), 1)
    tri = (c < r).astype(f32)
    run = jnp.zeros((1, EPAD), f32)
    for ch in range(U // ST):
        ind_c = ind_ref[ch * ST:(ch + 1) * ST, :]
        rank_c = jnp.dot(tri, ind_c, preferred_element_type=f32) + run
        rank_ref[ch * ST:(ch + 1) * ST, :] = rank_c
        run = run + jnp.sum(ind_c, axis=0, keepdims=True)
    counts = run
    re = lax.broadcasted_iota(jnp.int32, (EPAD, EPAD), 0)
    ce = lax.broadcasted_iota(jnp.int32, (EPAD, EPAD), 1)
    trie = (re < ce).astype(f32)
    offs = jnp.dot(counts, trie, preferred_element_type=f32)
    for ch in range(U // ST):
        ind_c = ind_ref[ch * ST:(ch + 1) * ST, :]
        d = jnp.sum(ind_c * (rank_ref[ch * ST:(ch + 1) * ST, :] + offs),
                    axis=1, keepdims=True)
        dest_ref[ch * ST:(ch + 1) * ST, :] = d.astype(jnp.int32)
    jv = lax.broadcasted_iota(jnp.int32, (NT, 1), 0).astype(f32)
    tile_lo = jv * ST
    tile_hi = tile_lo + ST
    lo_e = offs
    hi_e = offs + counts
    flag = ((lo_e < tile_hi) & (hi_e > tile_lo) & (counts > 0.0)).astype(f32)
    rowsum = jnp.sum(flag, axis=1, keepdims=True)
    rj = lax.broadcasted_iota(jnp.int32, (NT, NT), 0)
    cj = lax.broadcasted_iota(jnp.int32, (NT, NT), 1)
    trij = (cj < rj).astype(f32)
    prevrows = jnp.dot(trij, rowsum, preferred_element_type=f32)
    excl_e = jnp.dot(flag, trie, preferred_element_type=f32)
    widx = prevrows + excl_e
    first = flag * (excl_e == 0.0).astype(f32)
    ev = lax.broadcasted_iota(jnp.int32, (1, EPAD), 1).astype(f32)
    wlane = lax.broadcasted_iota(jnp.int32, (1, EPAD), 1)
    rows = []
    wt_row = jnp.zeros((1, EPAD), f32)
    we_row = jnp.zeros((1, EPAD), f32)
    wl_row = jnp.zeros((1, EPAD), f32)
    wh_row = jnp.zeros((1, EPAD), f32)
    wf_row = jnp.zeros((1, EPAD), f32)
    for w in range(NW):
        sel = flag * (widx == float(w)).astype(f32)
        has = jnp.sum(sel)
        wt = jnp.sum(sel * jv) + (1.0 - has) * float(NT - 1)
        we = jnp.sum(sel * ev)
        wl = jnp.sum(sel * jnp.maximum(lo_e, tile_lo))
        wh = jnp.sum(sel * jnp.minimum(hi_e, tile_hi))
        wf = jnp.sum(sel * first)
        oh = (wlane == w).astype(f32)
        wt_row = wt_row + oh * wt
        we_row = we_row + oh * we
        wl_row = wl_row + oh * wl
        wh_row = wh_row + oh * wh
        wf_row = wf_row + oh * wf
    z = jnp.zeros((1, EPAD), f32)
    meta = jnp.concatenate([wt_row, we_row, wl_row, wh_row, wf_row, z, z, z], axis=0)
    meta_ref[...] = meta.astype(jnp.int32)


def _group_kernel(wt_ref, we_ref, wl_ref, wh_ref, wf_ref, h_ref, w_ref, out_ref):
    w = pl.program_id(0)
    rows = wt_ref[w] * ST + lax.broadcasted_iota(jnp.int32, (ST, 1), 0)
    mask = ((rows >= wl_ref[w]) & (rows < wh_ref[w])).astype(jnp.float32)
    hm = h_ref[...] * mask
    y = lax.dot_general(hm, w_ref[0], (((1,), (1,)), ((), ())),
                        preferred_element_type=jnp.float32)

    @pl.when(wf_ref[w] == 1)
    def _():
        out_ref[...] = y

    @pl.when(wf_ref[w] != 1)
    def _():
        out_ref[...] += y


def _post_kernel(oa_ref, ob2_ref, inp_ref, tgt_ref, emb_ref, owt_ref, ob_ref, out_ref):
    i = pl.program_id(0)
    o = oa_ref[...] + ob2_ref[...]
    d, sc, sh = o[:, :F], o[:, F:2 * F], o[:, 2 * F:]
    r = lax.broadcasted_iota(jnp.int32, (F, F), 0)
    c = lax.broadcasted_iota(jnp.int32, (F, F), 1)
    tri = (r <= c).astype(jnp.float32)
    cum = jnp.dot(d, tri, preferred_element_type=jnp.float32)
    pos = (i * ST + lax.broadcasted_iota(jnp.int32, (ST, 1), 0)).astype(jnp.float32)
    y = cum / (pos + 1.0) * sc + sh
    y = y - jnp.mean(y, axis=1, keepdims=True)
    nrm = jnp.sqrt(jnp.sum(y * y, axis=1, keepdims=True))
    y = y / (nrm * (F ** -0.5) + 1e-5)
    col = inp_ref[...]
    lane = lax.broadcasted_iota(jnp.int32, (ST, C), 1)
    onehot = (col == lane).astype(jnp.float32)
    x = jnp.dot(onehot, emb_ref[...], preferred_element_type=jnp.float32)
    x0, x1 = x[:, :F], x[:, F:]
    y1 = x0 * BETA + y * (1.0 - BETA)
    y2 = x1 + y1
    cat = jnp.concatenate([y1, y2], axis=1)
    logits = jnp.dot(cat, owt_ref[...], preferred_element_type=jnp.float32) + ob_ref[...]
    m = jnp.max(logits, axis=1, keepdims=True)
    lse = m + jnp.log(jnp.sum(jnp.exp(logits - m), axis=1, keepdims=True))
    tcol = tgt_ref[...]
    tsel = (tcol == lane).astype(jnp.float32)
    g = jnp.sum(logits * tsel, axis=1, keepdims=True)
    part = jnp.sum(lse - g, keepdims=True).reshape(1, 1)

    @pl.when(i == 0)
    def _():
        out_ref[...] = jnp.zeros_like(out_ref)

    out_ref[...] += part


def _make_sc_kernels():
    mesh = plsc.VectorSubcoreMesh(core_axis_name="c", subcore_axis_name="s")
    nc, ns = mesh.num_cores, mesh.num_subcores
    nw = nc * ns
    i32, f32 = jnp.int32, jnp.float32

    g_rows = U // nw

    @functools.partial(
        pl.kernel, out_type=jax.ShapeDtypeStruct((U, I), f32), mesh=mesh,
        scratch_types=[pltpu.VMEM((g_rows,), i32), pltpu.VMEM((g_rows, I), f32),
                       pltpu.SemaphoreType.DMA])
    def dispatch_sc(ha_hbm, hb_hbm, dest_hbm, out_hbm, idx_v, rows_v, sem):
        wid = lax.axis_index("s") * nc + lax.axis_index("c")
        base = wid * g_rows
        pltpu.sync_copy(dest_hbm.at[pl.ds(base, g_rows)], idx_v)

        @pl.when(base < S)
        def _():
            pltpu.sync_copy(ha_hbm.at[pl.ds(base, g_rows)], rows_v)

        @pl.when(base >= S)
        def _():
            pltpu.sync_copy(hb_hbm.at[pl.ds(base - S, g_rows)], rows_v)

        pltpu.async_copy(rows_v, out_hbm.at[idx_v], sem).wait()

    u_per_w = U // nw
    CH = 32
    D3 = 3 * F

    @functools.partial(
        pl.kernel, out_type=jax.ShapeDtypeStruct((U, D3), f32), mesh=mesh,
        scratch_types=[pltpu.VMEM((u_per_w,), i32), pltpu.VMEM((CH, D3), f32),
                       pltpu.SemaphoreType.DMA])
    def combine_sc(ysort_hbm, dest_hbm, out_hbm, idx_v, rows_v, sem):
        wid = lax.axis_index("s") * nc + lax.axis_index("c")
        base = wid * u_per_w
        pltpu.sync_copy(dest_hbm.at[pl.ds(base, u_per_w)], idx_v)

        def chunk(ch, carry):
            pltpu.async_copy(
                ysort_hbm.at[idx_v.at[pl.ds(ch * CH, CH)]], rows_v, sem).wait()
            pltpu.sync_copy(rows_v, out_hbm.at[pl.ds(base + ch * CH, CH)])
            return carry

        lax.fori_loop(0, u_per_w // CH, chunk, 0)

    return dispatch_sc, combine_sc


def kernel(inp, tgt, emb, gate_w_in, gate_b_in, w_moe_in, w1, gate_w_out, gate_b_out, w_moe_out, out_w, out_b):
    f32, i32 = jnp.float32, jnp.int32
    inp2 = inp.reshape(S, 1).astype(i32)
    tgt2 = tgt.reshape(S, 1).astype(i32)
    emb_hi = emb[:, F:]
    gw_in = jnp.zeros((F, EPAD), f32).at[:, :E].set(gate_w_in)
    gb_in = jnp.zeros((1, EPAD), f32).at[0, :E].set(gate_b_in)
    gw_out = jnp.zeros((I, EPAD), f32).at[:, :E].set(gate_w_out)
    gb_out = jnp.zeros((1, EPAD), f32).at[0, :E].set(gate_b_out)
    wk = jnp.transpose(w1, (2, 0, 1))
    owt = out_w.T
    obr = out_b.reshape(1, C)

    h1 = pl.pallas_call(
        _moe_in_kernel,
        grid=(NS,),
        in_specs=[
            pl.BlockSpec((ST, 1), lambda i: (i, 0)),
            pl.BlockSpec((C, F), lambda i: (0, 0)),
            pl.BlockSpec((F, EPAD), lambda i: (0, 0)),
            pl.BlockSpec((1, EPAD), lambda i: (0, 0)),
            pl.BlockSpec((E, I, F), lambda i: (0, 0, 0)),
        ],
        out_specs=pl.BlockSpec((ST, I), lambda i: (i, 0)),
        out_shape=jax.ShapeDtypeStruct((S, I), f32),
    )(inp2, emb_hi, gw_in, gb_in, w_moe_in)

    h1p = jnp.zeros((S + 8, I), f32).at[8:].set(h1)

    ha, hb, dest2d, meta = pl.pallas_call(
        _conv_kernel,
        grid=(NS + 1,),
        in_specs=[
            pl.BlockSpec((S + 8, I), lambda i: (0, 0)),
            pl.BlockSpec((K, I, I), lambda i: (0, 0, 0)),
            pl.BlockSpec((I, EPAD), lambda i: (0, 0)),
            pl.BlockSpec((1, EPAD), lambda i: (0, 0)),
        ],
        out_specs=[
            pl.BlockSpec((ST, I), lambda i: (jnp.minimum(i, NS - 1), 0)),
            pl.BlockSpec((ST, I), lambda i: (jnp.minimum(i, NS - 1), 0)),
            pl.BlockSpec((U, 1), lambda i: (0, 0)),
            pl.BlockSpec((8, EPAD), lambda i: (0, 0)),
        ],
        out_shape=[
            jax.ShapeDtypeStruct((S, I), f32),
            jax.ShapeDtypeStruct((S, I), f32),
            jax.ShapeDtypeStruct((U, 1), i32),
            jax.ShapeDtypeStruct((8, EPAD), i32),
        ],
        scratch_shapes=[pltpu.VMEM((U, EPAD), f32), pltpu.VMEM((U, EPAD), f32)],
    )(h1p, wk, gw_out, gb_out)

    dest = dest2d.reshape(U)
    wt, we, wl, wh, wf = (meta[0, :NW], meta[1, :NW], meta[2, :NW],
                          meta[3, :NW], meta[4, :NW])

    dispatch_sc, combine_sc = _make_sc_kernels()
    hsort = dispatch_sc(ha, hb, dest)

    ysort = pl.pallas_call(
        _group_kernel,
        grid_spec=pltpu.PrefetchScalarGridSpec(
            num_scalar_prefetch=5,
            grid=(NW,),
            in_specs=[
                pl.BlockSpec((ST, I), lambda w, wt, we, wl, wh, wf: (wt[w], 0)),
                pl.BlockSpec((1, 3 * F, I), lambda w, wt, we, wl, wh, wf: (we[w], 0, 0)),
            ],
            out_specs=pl.BlockSpec(
                (ST, 3 * F), lambda w, wt, we, wl, wh, wf: (wt[w], 0)),
        ),
        out_shape=jax.ShapeDtypeStruct((U, 3 * F), f32),
    )(wt, we, wl, wh, wf, hsort, w_moe_out)

    o_slots = combine_sc(ysort, dest)

    tot = pl.pallas_call(
        _post_kernel,
        grid=(NS,),
        in_specs=[
            pl.BlockSpec((ST, 3 * F), lambda i: (i, 0)),
            pl.BlockSpec((ST, 3 * F), lambda i: (i + NS, 0)),
            pl.BlockSpec((ST, 1), lambda i: (i, 0)),
            pl.BlockSpec((ST, 1), lambda i: (i, 0)),
            pl.BlockSpec((C, 2 * F), lambda i: (0, 0)),
            pl.BlockSpec((2 * F, C), lambda i: (0, 0)),
            pl.BlockSpec((1, C), lambda i: (0, 0)),
        ],
        out_specs=pl.BlockSpec((1, 1), lambda i: (0, 0)),
        out_shape=jax.ShapeDtypeStruct((1, 1), f32),
    )(o_slots, o_slots, inp2, tgt2, emb, owt, obr)

    return tot[0, 0] / float(B * S)


# speedup vs baseline: 1.3794x; 1.0006x over previous
import functools

import jax
import jax.numpy as jnp
from jax import lax
from jax.experimental import pallas as pl
from jax.experimental.pallas import tpu as pltpu
from jax.experimental.pallas import tpu_sc as plsc

B, S, F, I, K, E, TOPK, C = 1, 2048, 768, 768, 5, 8, 2, 256
BETA = 0.5
ST = 256
NS = S // ST
EPAD = 128
U = 2 * S
NT = U // ST
NW = NT + E - 1


def _top2_parts(logits):
    lane = lax.broadcasted_iota(jnp.int32, logits.shape, 1)
    masked = jnp.where(lane < E, logits, -1e30)
    i1 = jnp.argmax(masked, axis=1, keepdims=True)
    v1 = jnp.max(masked, axis=1, keepdims=True)
    masked2 = jnp.where(lane == i1, -1e30, masked)
    i2 = jnp.argmax(masked2, axis=1, keepdims=True)
    v2 = jnp.max(masked2, axis=1, keepdims=True)
    s1 = jax.nn.sigmoid(v1 - v2)
    oh1 = (lane == i1).astype(jnp.float32)
    oh2 = (lane == i2).astype(jnp.float32)
    return oh1, oh2, s1, 1.0 - s1


def _moe_in_kernel(inp_ref, emb_hi_ref, gw_ref, gb_ref, w_ref, out_ref):
    col = inp_ref[...]
    lane = lax.broadcasted_iota(jnp.int32, (ST, C), 1)
    onehot = (col == lane).astype(jnp.float32)
    h = jnp.dot(onehot, emb_hi_ref[...], preferred_element_type=jnp.float32)
    logits = jnp.dot(h, gw_ref[...], preferred_element_type=jnp.float32) + gb_ref[...]
    oh1, oh2, s1, s2 = _top2_parts(logits)
    comb = oh1 * s1 + oh2 * s2
    acc = jnp.zeros((ST, I), jnp.float32)
    for e in range(E):
        ye = lax.dot_general(h, w_ref[e], (((1,), (1,)), ((), ())),
                             preferred_element_type=jnp.float32)
        acc = acc + comb[:, e:e + 1] * ye
    out_ref[...] = jnp.maximum(acc, 0.0)


def _conv_kernel(h1p_ref, wk_ref, gw_ref, gb_ref, ha_ref, hb_ref, dest_ref, meta_ref, ind_ref, rank_ref):
    i = pl.program_id(0)

    @pl.when(i < NS)
    def _():
        im = jnp.minimum(i, NS - 1)
        blk = h1p_ref[pl.ds(im * ST, ST + 8), :]
        acc = jnp.zeros((ST, I), jnp.float32)
        for kk in range(K):
            xs = lax.slice(blk, (4 + kk, 0), (4 + kk + ST, I))
            acc = acc + lax.dot_general(xs, wk_ref[kk], (((1,), (1,)), ((), ())),
                                        preferred_element_type=jnp.float32)
        h2 = jnp.maximum(acc, 0.0)
        logits = jnp.dot(h2, gw_ref[...], preferred_element_type=jnp.float32) + gb_ref[...]
        oh1, oh2, s1, s2 = _top2_parts(logits)
        ha_ref[...] = h2 * s1
        hb_ref[...] = h2 * s2
        ind_ref[pl.ds(im * ST, ST), :] = oh1
        ind_ref[pl.ds(S + im * ST, ST), :] = oh2

    @pl.when(i == NS)
    def _():
        _route(ind_ref, dest_ref, meta_ref, rank_ref)


def _route(ind_ref, dest_ref, meta_ref, rank_ref):
    f32 = jnp.float32
    r = lax.broadcasted_iota(jnp.int32, (ST, ST), 0)
    c = lax.broadcasted_iota(jnp.int32, (ST, ST), 1)
    tri = (c < r).astype(f32)
    run = jnp.zeros((1, EPAD), f32)
    for ch in range(U // ST):
        ind_c = ind_ref[ch * ST:(ch + 1) * ST, :]
        rank_c = jnp.dot(tri, ind_c, preferred_element_type=f32) + run
        rank_ref[ch * ST:(ch + 1) * ST, :] = rank_c
        run = run + jnp.sum(ind_c, axis=0, keepdims=True)
    counts = run
    re = lax.broadcasted_iota(jnp.int32, (EPAD, EPAD), 0)
    ce = lax.broadcasted_iota(jnp.int32, (EPAD, EPAD), 1)
    trie = (re < ce).astype(f32)
    offs = jnp.dot(counts, trie, preferred_element_type=f32)
    for ch in range(U // ST):
        ind_c = ind_ref[ch * ST:(ch + 1) * ST, :]
        d = jnp.sum(ind_c * (rank_ref[ch * ST:(ch + 1) * ST, :] + offs),
                    axis=1, keepdims=True)
        dest_ref[ch * ST:(ch + 1) * ST, :] = d.astype(jnp.int32)
    jv = lax.broadcasted_iota(jnp.int32, (NT, 1), 0).astype(f32)
    tile_lo = jv * ST
    tile_hi = tile_lo + ST
    lo_e = offs
    hi_e = offs + counts
    flag = ((lo_e < tile_hi) & (hi_e > tile_lo) & (counts > 0.0)).astype(f32)
    rowsum = jnp.sum(flag, axis=1, keepdims=True)
    rj = lax.broadcasted_iota(jnp.int32, (NT, NT), 0)
    cj = lax.broadcasted_iota(jnp.int32, (NT, NT), 1)
    trij = (cj < rj).astype(f32)
    prevrows = jnp.dot(trij, rowsum, preferred_element_type=f32)
    excl_e = jnp.dot(flag, trie, preferred_element_type=f32)
    widx = prevrows + excl_e
    first = flag * (excl_e == 0.0).astype(f32)
    ev = lax.broadcasted_iota(jnp.int32, (1, EPAD), 1).astype(f32)
    wlane = lax.broadcasted_iota(jnp.int32, (1, EPAD), 1)
    wt_row = jnp.zeros((1, EPAD), f32)
    we_row = jnp.zeros((1, EPAD), f32)
    wl_row = jnp.zeros((1, EPAD), f32)
    wh_row = jnp.zeros((1, EPAD), f32)
    wf_row = jnp.zeros((1, EPAD), f32)
    for w in range(NW):
        sel = flag * (widx == float(w)).astype(f32)
        has = jnp.sum(sel)
        wt = jnp.sum(sel * jv) + (1.0 - has) * float(NT - 1)
        we = jnp.sum(sel * ev)
        wl = jnp.sum(sel * jnp.maximum(lo_e, tile_lo))
        wh = jnp.sum(sel * jnp.minimum(hi_e, tile_hi))
        wf = jnp.sum(sel * first)
        oh = (wlane == w).astype(f32)
        wt_row = wt_row + oh * wt
        we_row = we_row + oh * we
        wl_row = wl_row + oh * wl
        wh_row = wh_row + oh * wh
        wf_row = wf_row + oh * wf
    z = jnp.zeros((1, EPAD), f32)
    meta = jnp.concatenate([wt_row, we_row, wl_row, wh_row, wf_row, z, z, z], axis=0)
    meta_ref[...] = meta.astype(jnp.int32)


def _group_kernel(wt_ref, we_ref, wl_ref, wh_ref, wf_ref, h_ref, w_ref, out_ref):
    w = pl.program_id(0)
    rows = wt_ref[w] * ST + lax.broadcasted_iota(jnp.int32, (ST, 1), 0)
    mask = ((rows >= wl_ref[w]) & (rows < wh_ref[w])).astype(jnp.float32)
    hm = h_ref[...] * mask
    y = lax.dot_general(hm, w_ref[0], (((1,), (1,)), ((), ())),
                        preferred_element_type=jnp.float32)

    @pl.when(wf_ref[w] == 1)
    def _():
        out_ref[...] = y

    @pl.when(wf_ref[w] != 1)
    def _():
        out_ref[...] += y


def _post_kernel(oa_ref, ob2_ref, inp_ref, tgt_ref, emb_ref, owt_ref, ob_ref, out_ref):
    i = pl.program_id(0)
    o = oa_ref[...] + ob2_ref[...]
    d, sc, sh = o[:, :F], o[:, F:2 * F], o[:, 2 * F:]
    r = lax.broadcasted_iota(jnp.int32, (F, F), 0)
    c = lax.broadcasted_iota(jnp.int32, (F, F), 1)
    tri = (r <= c).astype(jnp.float32)
    cum = jnp.dot(d, tri, preferred_element_type=jnp.float32)
    pos = (i * ST + lax.broadcasted_iota(jnp.int32, (ST, 1), 0)).astype(jnp.float32)
    y = cum / (pos + 1.0) * sc + sh
    y = y - jnp.mean(y, axis=1, keepdims=True)
    nrm = jnp.sqrt(jnp.sum(y * y, axis=1, keepdims=True))
    y = y / (nrm * (F ** -0.5) + 1e-5)
    col = inp_ref[...]
    lane = lax.broadcasted_iota(jnp.int32, (ST, C), 1)
    onehot = (col == lane).astype(jnp.float32)
    x = jnp.dot(onehot, emb_ref[...], preferred_element_type=jnp.float32)
    x0, x1 = x[:, :F], x[:, F:]
    y1 = x0 * BETA + y * (1.0 - BETA)
    y2 = x1 + y1
    cat = jnp.concatenate([y1, y2], axis=1)
    logits = jnp.dot(cat, owt_ref[...], preferred_element_type=jnp.float32) + ob_ref[...]
    m = jnp.max(logits, axis=1, keepdims=True)
    lse = m + jnp.log(jnp.sum(jnp.exp(logits - m), axis=1, keepdims=True))
    tcol = tgt_ref[...]
    tsel = (tcol == lane).astype(jnp.float32)
    g = jnp.sum(logits * tsel, axis=1, keepdims=True)
    part = jnp.sum(lse - g, keepdims=True).reshape(1, 1)

    @pl.when(i == 0)
    def _():
        out_ref[...] = jnp.zeros_like(out_ref)

    out_ref[...] += part


def _make_sc_kernels():
    mesh = plsc.VectorSubcoreMesh(core_axis_name="c", subcore_axis_name="s")
    nc, ns = mesh.num_cores, mesh.num_subcores
    nw = nc * ns
    i32, f32 = jnp.int32, jnp.float32

    g_rows = U // nw

    @functools.partial(
        pl.kernel, out_type=jax.ShapeDtypeStruct((U, I), f32), mesh=mesh,
        scratch_types=[pltpu.VMEM((g_rows,), i32), pltpu.VMEM((g_rows, I), f32),
                       pltpu.SemaphoreType.DMA])
    def dispatch_sc(ha_hbm, hb_hbm, dest_hbm, out_hbm, idx_v, rows_v, sem):
        wid = lax.axis_index("s") * nc + lax.axis_index("c")
        base = wid * g_rows
        pltpu.sync_copy(dest_hbm.at[pl.ds(base, g_rows)], idx_v)

        @pl.when(base < S)
        def _():
            pltpu.sync_copy(ha_hbm.at[pl.ds(base, g_rows)], rows_v)

        @pl.when(base >= S)
        def _():
            pltpu.sync_copy(hb_hbm.at[pl.ds(base - S, g_rows)], rows_v)

        pltpu.async_copy(rows_v, out_hbm.at[idx_v], sem).wait()

    u_per_w = U // nw
    CH = 32
    D3 = 3 * F

    @functools.partial(
        pl.kernel, out_type=jax.ShapeDtypeStruct((U, D3), f32), mesh=mesh,
        scratch_types=[pltpu.VMEM((u_per_w,), i32), pltpu.VMEM((CH, D3), f32),
                       pltpu.SemaphoreType.DMA])
    def combine_sc(ysort_hbm, dest_hbm, out_hbm, idx_v, rows_v, sem):
        wid = lax.axis_index("s") * nc + lax.axis_index("c")
        base = wid * u_per_w
        pltpu.sync_copy(dest_hbm.at[pl.ds(base, u_per_w)], idx_v)

        def chunk(ch, carry):
            pltpu.async_copy(
                ysort_hbm.at[idx_v.at[pl.ds(ch * CH, CH)]], rows_v, sem).wait()
            pltpu.sync_copy(rows_v, out_hbm.at[pl.ds(base + ch * CH, CH)])
            return carry

        lax.fori_loop(0, u_per_w // CH, chunk, 0)

    return dispatch_sc, combine_sc


def kernel(inp, tgt, emb, gate_w_in, gate_b_in, w_moe_in, w1, gate_w_out, gate_b_out, w_moe_out, out_w, out_b):
    f32, i32 = jnp.float32, jnp.int32
    inp2 = inp.reshape(S, 1).astype(i32)
    tgt2 = tgt.reshape(S, 1).astype(i32)
    emb_hi = emb[:, F:]
    gw_in = jnp.zeros((F, EPAD), f32).at[:, :E].set(gate_w_in)
    gb_in = jnp.zeros((1, EPAD), f32).at[0, :E].set(gate_b_in)
    gw_out = jnp.zeros((I, EPAD), f32).at[:, :E].set(gate_w_out)
    gb_out = jnp.zeros((1, EPAD), f32).at[0, :E].set(gate_b_out)
    wk = jnp.transpose(w1, (2, 0, 1))
    owt = out_w.T
    obr = out_b.reshape(1, C)

    h1 = pl.pallas_call(
        _moe_in_kernel,
        grid=(NS,),
        in_specs=[
            pl.BlockSpec((ST, 1), lambda i: (i, 0)),
            pl.BlockSpec((C, F), lambda i: (0, 0)),
            pl.BlockSpec((F, EPAD), lambda i: (0, 0)),
            pl.BlockSpec((1, EPAD), lambda i: (0, 0)),
            pl.BlockSpec((E, I, F), lambda i: (0, 0, 0)),
        ],
        out_specs=pl.BlockSpec((ST, I), lambda i: (i, 0)),
        out_shape=jax.ShapeDtypeStruct((S, I), f32),
    )(inp2, emb_hi, gw_in, gb_in, w_moe_in)

    h1p = jnp.zeros((S + 8, I), f32).at[8:].set(h1)

    ha, hb, dest2d, meta = pl.pallas_call(
        _conv_kernel,
        grid=(NS + 1,),
        in_specs=[
            pl.BlockSpec((S + 8, I), lambda i: (0, 0)),
            pl.BlockSpec((K, I, I), lambda i: (0, 0, 0)),
            pl.BlockSpec((I, EPAD), lambda i: (0, 0)),
            pl.BlockSpec((1, EPAD), lambda i: (0, 0)),
        ],
        out_specs=[
            pl.BlockSpec((ST, I), lambda i: (jnp.minimum(i, NS - 1), 0)),
            pl.BlockSpec((ST, I), lambda i: (jnp.minimum(i, NS - 1), 0)),
            pl.BlockSpec((U, 1), lambda i: (0, 0)),
            pl.BlockSpec((8, EPAD), lambda i: (0, 0)),
        ],
        out_shape=[
            jax.ShapeDtypeStruct((S, I), f32),
            jax.ShapeDtypeStruct((S, I), f32),
            jax.ShapeDtypeStruct((U, 1), i32),
            jax.ShapeDtypeStruct((8, EPAD), i32),
        ],
        scratch_shapes=[pltpu.VMEM((U, EPAD), f32), pltpu.VMEM((U, EPAD), f32)],
    )(h1p, wk, gw_out, gb_out)

    dest = dest2d.reshape(U)
    wt, we, wl, wh, wf = (meta[0, :NW], meta[1, :NW], meta[2, :NW],
                          meta[3, :NW], meta[4, :NW])

    dispatch_sc, combine_sc = _make_sc_kernels()
    hsort = dispatch_sc(ha, hb, dest)

    ysort = pl.pallas_call(
        _group_kernel,
        grid_spec=pltpu.PrefetchScalarGridSpec(
            num_scalar_prefetch=5,
            grid=(NW,),
            in_specs=[
                pl.BlockSpec((ST, I), lambda w, wt, we, wl, wh, wf: (wt[w], 0)),
                pl.BlockSpec((1, 3 * F, I), lambda w, wt, we, wl, wh, wf: (we[w], 0, 0)),
            ],
            out_specs=pl.BlockSpec(
                (ST, 3 * F), lambda w, wt, we, wl, wh, wf: (wt[w], 0)),
        ),
        out_shape=jax.ShapeDtypeStruct((U, 3 * F), f32),
    )(wt, we, wl, wh, wf, hsort, w_moe_out)

    o_slots = combine_sc(ysort, dest)

    tot = pl.pallas_call(
        _post_kernel,
        grid=(NS,),
        in_specs=[
            pl.BlockSpec((ST, 3 * F), lambda i: (i, 0)),
            pl.BlockSpec((ST, 3 * F), lambda i: (i + NS, 0)),
            pl.BlockSpec((ST, 1), lambda i: (i, 0)),
            pl.BlockSpec((ST, 1), lambda i: (i, 0)),
            pl.BlockSpec((C, 2 * F), lambda i: (0, 0)),
            pl.BlockSpec((2 * F, C), lambda i: (0, 0)),
            pl.BlockSpec((1, C), lambda i: (0, 0)),
        ],
        out_specs=pl.BlockSpec((1, 1), lambda i: (0, 0)),
        out_shape=jax.ShapeDtypeStruct((1, 1), f32),
    )(o_slots, o_slots, inp2, tgt2, emb, owt, obr)

    return tot[0, 0] / float(B * S)
